# baseline (speedup 1.0000x reference)
import functools

import jax
import jax.numpy as jnp
from jax import lax
from jax.experimental import pallas as pl
from jax.experimental.pallas import tpu as pltpu
from jax.experimental.pallas import tpu_sc as plsc

E = 8
H = 1024
F = 768
N = 2048
K = 2
BM = 256
NB = (N * K) // BM + (E - 1)
M = NB * BM
NW = 32
TPW = N // NW


def _router_body(x_ref, rw_ref, pos_ref, wr_ref, be_ref):
    xb = x_ref[...].astype(jnp.bfloat16)
    logits = lax.dot_general(
        xb, rw_ref[...].astype(jnp.bfloat16), (((1,), (1,)), ((), ())),
        preferred_element_type=jnp.float32)
    i8 = lax.broadcasted_iota(jnp.int32, (N, E), 1)
    m1 = jnp.max(logits, axis=1, keepdims=True)
    am1 = jnp.min(jnp.where(logits == m1, i8, E), axis=1, keepdims=True)
    lm = jnp.where(i8 == am1, -jnp.inf, logits)
    m2 = jnp.max(lm, axis=1, keepdims=True)
    am2 = jnp.min(jnp.where(lm == m2, i8, E), axis=1, keepdims=True)
    w1 = 1.0 / (1.0 + jnp.exp(m2 - m1))
    w2 = 1.0 - w1

    exp0 = am1[:, 0].reshape(16, 128)
    exp1 = am2[:, 0].reshape(16, 128)
    expc = jnp.concatenate([exp0, exp1], axis=0)
    eidx = lax.broadcasted_iota(jnp.int32, (32, E, 128), 1)
    ohv = (expc[:, None, :] == eidx).astype(jnp.float32)

    oh2 = ohv.reshape(32 * E, 128)
    xi = lax.broadcasted_iota(jnp.int32, (128, 128), 0)
    yi = lax.broadcasted_iota(jnp.int32, (128, 128), 1)
    tri = (yi < xi).astype(jnp.bfloat16)
    within = lax.dot_general(
        oh2.astype(jnp.bfloat16), tri, (((1,), (1,)), ((), ())),
        preferred_element_type=jnp.float32).reshape(32, E, 128)
    s = jnp.sum(ohv, axis=2)
    bi = lax.broadcasted_iota(jnp.int32, (32, 32), 0)
    bj = lax.broadcasted_iota(jnp.int32, (32, 32), 1)
    bmask = (bj < bi).astype(jnp.float32)
    excl = jnp.sum(bmask[:, :, None] * s[None, :, :], axis=1)

    counts = jnp.sum(s, axis=0).astype(jnp.int32)
    nblk = lax.shift_right_logical(counts + (BM - 1), 8)
    pc = lax.shift_left(nblk, 8).astype(jnp.float32)
    ei = lax.broadcasted_iota(jnp.int32, (E, E), 0)
    ej = lax.broadcasted_iota(jnp.int32, (E, E), 1)
    emask = (ej < ei).astype(jnp.float32)
    offs = jnp.sum(emask * pc[None, :], axis=1)

    slot = jnp.sum(
        ohv * (offs[None, :, None] + excl[:, :, None] + within), axis=1)
    pos_ref[...] = slot.astype(jnp.int32)
    w1b = jnp.broadcast_to(w1, (N, 128))
    w2b = jnp.broadcast_to(w2, (N, 128))
    wr_ref[...] = jnp.concatenate(
        [w1b[None], w2b[None]], axis=0)
    cumi = jnp.sum(jnp.where(emask > 0, nblk[None, :], 0), axis=1) + nblk
    bidx = lax.broadcasted_iota(jnp.int32, (NB, E), 0)
    be = jnp.minimum(jnp.sum((cumi[None, :] <= bidx).astype(jnp.int32),
                             axis=1), E - 1)
    be_ref[...] = be.reshape(1, NB)


def _router(x, rw):
    return pl.pallas_call(
        _router_body,
        out_shape=(
            jax.ShapeDtypeStruct((NW, 128), jnp.int32),
            jax.ShapeDtypeStruct((K, N, 128), jnp.float32),
            jax.ShapeDtypeStruct((1, NB), jnp.int32),
        ),
    )(x, rw)


def _dispatch(x, wr, pos):
    mesh = plsc.VectorSubcoreMesh(core_axis_name="c", subcore_axis_name="s")
    CH = 64

    @functools.partial(
        pl.kernel,
        out_type=(
            jax.ShapeDtypeStruct((M, H), jnp.float32),
            jax.ShapeDtypeStruct((M, 128), jnp.float32),
        ),
        mesh=mesh,
        scratch_types=[
            pltpu.VMEM((CH,), jnp.int32),
            pltpu.VMEM((CH,), jnp.int32),
            pltpu.VMEM((CH, H), jnp.float32),
            pltpu.VMEM((128, 128), jnp.float32),
        ],
    )
    def k(x_hbm, wr_hbm, pos_hbm, xs_hbm, ws_hbm, ia_v, ib_v, x_v, w_v):
        wid = lax.axis_index("s") * 2 + lax.axis_index("c")
        kk = wid // 16
        row = wid % 16
        tb = row * 128
        pltpu.sync_copy(pos_hbm.at[wid, pl.ds(0, CH)], ia_v)
        pltpu.sync_copy(pos_hbm.at[wid, pl.ds(CH, CH)], ib_v)
        pltpu.sync_copy(x_hbm.at[pl.ds(tb, CH)], x_v)
        pltpu.sync_copy(x_v, xs_hbm.at[ia_v])
        pltpu.sync_copy(x_hbm.at[pl.ds(tb + CH, CH)], x_v)
        pltpu.sync_copy(x_v, xs_hbm.at[ib_v])
        pltpu.sync_copy(wr_hbm.at[kk, pl.ds(tb, 128)], w_v)
        pltpu.sync_copy(w_v.at[pl.ds(0, CH)], ws_hbm.at[ia_v])
        pltpu.sync_copy(w_v.at[pl.ds(CH, CH)], ws_hbm.at[ib_v])

    return k(x, wr, pos)


def _experts_body(be_ref, x_ref, g_ref, u_ref, d_ref, w_ref, y_ref):
    xb = x_ref[...].astype(jnp.bfloat16)
    g = lax.dot_general(xb, g_ref[0], (((1,), (1,)), ((), ())),
                        preferred_element_type=jnp.float32)
    u = lax.dot_general(xb, u_ref[0], (((1,), (1,)), ((), ())),
                        preferred_element_type=jnp.float32)
    h = (g * lax.logistic(g) * u).astype(jnp.bfloat16)
    y = lax.dot_general(h, d_ref[0], (((1,), (1,)), ((), ())),
                        preferred_element_type=jnp.float32)
    y_ref[...] = y * w_ref[:, 0:1]


def _experts(block_expert, xs2, gate, up, down, ws):
    grid_spec = pltpu.PrefetchScalarGridSpec(
        num_scalar_prefetch=1,
        grid=(NB,),
        in_specs=[
            pl.BlockSpec((BM, H), lambda i, be: (i, 0)),
            pl.BlockSpec((1, F, H), lambda i, be: (be[0, i], 0, 0)),
            pl.BlockSpec((1, F, H), lambda i, be: (be[0, i], 0, 0)),
            pl.BlockSpec((1, H, F), lambda i, be: (be[0, i], 0, 0)),
            pl.BlockSpec((BM, 128), lambda i, be: (i, 0)),
        ],
        out_specs=pl.BlockSpec((BM, H), lambda i, be: (i, 0)),
    )
    return pl.pallas_call(
        _experts_body,
        grid_spec=grid_spec,
        out_shape=jax.ShapeDtypeStruct((M, H), jnp.float32),
        compiler_params=pltpu.CompilerParams(
            dimension_semantics=("arbitrary",),
        ),
    )(block_expert, xs2, gate, up, down, ws)


def _combine(ys, pos):
    mesh = plsc.VectorSubcoreMesh(core_axis_name="c", subcore_axis_name="s")
    CH = 16
    NC = TPW // CH

    def k(ys_hbm, pos_hbm, out_hbm, i0_v, i1_v, b0a_v, b1a_v, b0b_v, b1b_v,
          sem_a, sem_b, sem_o):
        wid = lax.axis_index("s") * 2 + lax.axis_index("c")
        prow = wid // 2
        pcol = (wid % 2) * TPW
        pltpu.sync_copy(pos_hbm.at[prow, pl.ds(pcol, TPW)], i0_v)
        pltpu.sync_copy(pos_hbm.at[16 + prow, pl.ds(pcol, TPW)], i1_v)
        bufs = ((b0a_v, b1a_v, sem_a), (b0b_v, b1b_v, sem_b))

        def start(c, b0, b1, sem):
            pltpu.async_copy(ys_hbm.at[i0_v.at[pl.ds(c * CH, CH)]], b0, sem)
            pltpu.async_copy(ys_hbm.at[i1_v.at[pl.ds(c * CH, CH)]], b1, sem)

        start(0, *bufs[0])
        for c in range(NC):
            b0, b1, sem = bufs[c % 2]
            pltpu.make_async_copy(ys_hbm.at[pl.ds(0, CH)], b0, sem).wait()
            pltpu.make_async_copy(ys_hbm.at[pl.ds(0, CH)], b1, sem).wait()
            if c > 0:
                pltpu.make_async_copy(
                    bufs[(c - 1) % 2][0],
                    out_hbm.at[pl.ds(wid * TPW + (c - 1) * CH, CH)],
                    sem_o).wait()
            if c + 1 < NC:
                start(c + 1, *bufs[(c + 1) % 2])

            @pl.loop(0, CH)
            def _(r):
                for j in range(0, H, 16):
                    b0[r, pl.ds(j, 16)] = (
                        b0[r, pl.ds(j, 16)] + b1[r, pl.ds(j, 16)])

            pltpu.async_copy(
                b0, out_hbm.at[pl.ds(wid * TPW + c * CH, CH)], sem_o)
        pltpu.make_async_copy(
            bufs[(NC - 1) % 2][0],
            out_hbm.at[pl.ds(wid * TPW + (NC - 1) * CH, CH)], sem_o).wait()

    kk = pl.kernel(
        k,
        out_type=jax.ShapeDtypeStruct((N, H), jnp.float32),
        mesh=mesh,
        scratch_types=[
            pltpu.VMEM((TPW,), jnp.int32),
            pltpu.VMEM((TPW,), jnp.int32),
            pltpu.VMEM((CH, H), jnp.float32),
            pltpu.VMEM((CH, H), jnp.float32),
            pltpu.VMEM((CH, H), jnp.float32),
            pltpu.VMEM((CH, H), jnp.float32),
            pltpu.SemaphoreType.DMA,
            pltpu.SemaphoreType.DMA,
            pltpu.SemaphoreType.DMA,
        ],
    )
    return kk(ys, pos)


def kernel(hidden_states, router_weight, gate_proj, up_proj, down_proj):
    B, T, Hc = hidden_states.shape
    x = hidden_states.reshape(-1, Hc)
    pos, wr, be = _router(x, router_weight)
    xs, ws = _dispatch(x, wr, pos)
    ys = _experts(be, xs, gate_proj.astype(jnp.bfloat16),
                  up_proj.astype(jnp.bfloat16),
                  down_proj.astype(jnp.bfloat16), ws)
    out = _combine(ys, pos)
    return out.reshape(B, T, Hc)


# speedup vs baseline: 1.1530x; 1.1530x over previous
import functools

import jax
import jax.numpy as jnp
from jax import lax
from jax.experimental import pallas as pl
from jax.experimental.pallas import tpu as pltpu
from jax.experimental.pallas import tpu_sc as plsc

E = 8
H = 1024
F = 768
N = 2048
K = 2
BM = 256
NB = (N * K) // BM + (E - 1)
M = NB * BM
NW = 32
TPW = N // NW


def _router_body(x_ref, rw_ref, pos_ref, wr_ref, be_ref):
    xb = x_ref[...].astype(jnp.bfloat16)
    logits = lax.dot_general(
        xb, rw_ref[...].astype(jnp.bfloat16), (((1,), (1,)), ((), ())),
        preferred_element_type=jnp.float32)
    i8 = lax.broadcasted_iota(jnp.int32, (N, E), 1)
    m1 = jnp.max(logits, axis=1, keepdims=True)
    am1 = jnp.min(jnp.where(logits == m1, i8, E), axis=1, keepdims=True)
    lm = jnp.where(i8 == am1, -jnp.inf, logits)
    m2 = jnp.max(lm, axis=1, keepdims=True)
    am2 = jnp.min(jnp.where(lm == m2, i8, E), axis=1, keepdims=True)
    w1 = 1.0 / (1.0 + jnp.exp(m2 - m1))
    w2 = 1.0 - w1

    exp0 = am1[:, 0].reshape(16, 128)
    exp1 = am2[:, 0].reshape(16, 128)
    expc = jnp.concatenate([exp0, exp1], axis=0)
    eidx = lax.broadcasted_iota(jnp.int32, (32, E, 128), 1)
    ohv = (expc[:, None, :] == eidx).astype(jnp.float32)

    oh2 = ohv.reshape(32 * E, 128)
    xi = lax.broadcasted_iota(jnp.int32, (128, 128), 0)
    yi = lax.broadcasted_iota(jnp.int32, (128, 128), 1)
    tri = (yi < xi).astype(jnp.bfloat16)
    within = lax.dot_general(
        oh2.astype(jnp.bfloat16), tri, (((1,), (1,)), ((), ())),
        preferred_element_type=jnp.float32).reshape(32, E, 128)
    s = jnp.sum(ohv, axis=2)
    bi = lax.broadcasted_iota(jnp.int32, (32, 32), 0)
    bj = lax.broadcasted_iota(jnp.int32, (32, 32), 1)
    bmask = (bj < bi).astype(jnp.float32)
    excl = jnp.sum(bmask[:, :, None] * s[None, :, :], axis=1)

    counts = jnp.sum(s, axis=0).astype(jnp.int32)
    nblk = lax.shift_right_logical(counts + (BM - 1), 8)
    pc = lax.shift_left(nblk, 8).astype(jnp.float32)
    ei = lax.broadcasted_iota(jnp.int32, (E, E), 0)
    ej = lax.broadcasted_iota(jnp.int32, (E, E), 1)
    emask = (ej < ei).astype(jnp.float32)
    offs = jnp.sum(emask * pc[None, :], axis=1)

    slot = jnp.sum(
        ohv * (offs[None, :, None] + excl[:, :, None] + within), axis=1)
    pos_ref[...] = slot.astype(jnp.int32)
    w1b = jnp.broadcast_to(w1, (N, 128))
    w2b = jnp.broadcast_to(w2, (N, 128))
    wr_ref[...] = jnp.concatenate(
        [w1b[None], w2b[None]], axis=0)
    cumi = jnp.sum(jnp.where(emask > 0, nblk[None, :], 0), axis=1) + nblk
    bidx = lax.broadcasted_iota(jnp.int32, (NB, E), 0)
    be = jnp.minimum(jnp.sum((cumi[None, :] <= bidx).astype(jnp.int32),
                             axis=1), E - 1)
    be_ref[...] = be.reshape(1, NB)


def _router(x, rw):
    return pl.pallas_call(
        _router_body,
        out_shape=(
            jax.ShapeDtypeStruct((NW, 128), jnp.int32),
            jax.ShapeDtypeStruct((K, N, 128), jnp.float32),
            jax.ShapeDtypeStruct((1, NB), jnp.int32),
        ),
    )(x, rw)


def _dispatch(x, wr, pos):
    mesh = plsc.VectorSubcoreMesh(core_axis_name="c", subcore_axis_name="s")
    CH = 32
    NC = 128 // CH

    def k(x_hbm, wr_hbm, pos_hbm, xs_hbm, ws_hbm,
          i0_v, i1_v, i2_v, i3_v, xa_v, xb_v, w_v,
          sem_la, sem_lb, sem_sa, sem_sb, sem_w):
        wid = lax.axis_index("s") * 2 + lax.axis_index("c")
        kk = wid // 16
        row = wid % 16
        tb = row * 128
        idxs = (i0_v, i1_v, i2_v, i3_v)
        for c in range(NC):
            pltpu.sync_copy(pos_hbm.at[wid, pl.ds(c * CH, CH)], idxs[c])
        pltpu.sync_copy(wr_hbm.at[kk, pl.ds(tb, 128)], w_v)
        for c in range(NC):
            pltpu.async_copy(
                w_v.at[pl.ds(c * CH, CH)], ws_hbm.at[idxs[c]], sem_w)
        bufs = ((xa_v, sem_la, sem_sa), (xb_v, sem_lb, sem_sb))
        pltpu.async_copy(x_hbm.at[pl.ds(tb, CH)], xa_v, sem_la)
        pltpu.async_copy(x_hbm.at[pl.ds(tb + CH, CH)], xb_v, sem_lb)
        for c in range(NC):
            xbuf, semL, semS = bufs[c % 2]
            pltpu.make_async_copy(
                x_hbm.at[pl.ds(0, CH)], xbuf, semL).wait()
            pltpu.async_copy(xbuf, xs_hbm.at[idxs[c]], semS)
            if c + 2 < NC:
                pltpu.make_async_copy(
                    xbuf, xs_hbm.at[idxs[c]], semS).wait()
                pltpu.async_copy(
                    x_hbm.at[pl.ds(tb + (c + 2) * CH, CH)], xbuf, semL)
        pltpu.make_async_copy(xa_v, xs_hbm.at[i0_v], sem_sa).wait()
        pltpu.make_async_copy(xb_v, xs_hbm.at[i1_v], sem_sb).wait()
        for _ in range(NC):
            pltpu.make_async_copy(
                w_v.at[pl.ds(0, CH)], ws_hbm.at[i0_v], sem_w).wait()

    kk2 = pl.kernel(
        k,
        out_type=(
            jax.ShapeDtypeStruct((M, H), jnp.float32),
            jax.ShapeDtypeStruct((M, 128), jnp.float32),
        ),
        mesh=mesh,
        scratch_types=[
            pltpu.VMEM((CH,), jnp.int32),
            pltpu.VMEM((CH,), jnp.int32),
            pltpu.VMEM((CH,), jnp.int32),
            pltpu.VMEM((CH,), jnp.int32),
            pltpu.VMEM((CH, H), jnp.float32),
            pltpu.VMEM((CH, H), jnp.float32),
            pltpu.VMEM((128, 128), jnp.float32),
            pltpu.SemaphoreType.DMA,
            pltpu.SemaphoreType.DMA,
            pltpu.SemaphoreType.DMA,
            pltpu.SemaphoreType.DMA,
            pltpu.SemaphoreType.DMA,
        ],
    )
    return kk2(x, wr, pos)


def _experts_body(be_ref, x_ref, g_ref, u_ref, d_ref, w_ref, y_ref):
    xb = x_ref[...].astype(jnp.bfloat16)
    g = lax.dot_general(xb, g_ref[0].astype(jnp.bfloat16),
                        (((1,), (1,)), ((), ())),
                        preferred_element_type=jnp.float32)
    u = lax.dot_general(xb, u_ref[0].astype(jnp.bfloat16),
                        (((1,), (1,)), ((), ())),
                        preferred_element_type=jnp.float32)
    h = (g * lax.logistic(g) * u).astype(jnp.bfloat16)
    y = lax.dot_general(h, d_ref[0].astype(jnp.bfloat16),
                        (((1,), (1,)), ((), ())),
                        preferred_element_type=jnp.float32)
    y_ref[...] = y * w_ref[:, 0:1]


def _experts(block_expert, xs2, gate, up, down, ws):
    grid_spec = pltpu.PrefetchScalarGridSpec(
        num_scalar_prefetch=1,
        grid=(NB,),
        in_specs=[
            pl.BlockSpec((BM, H), lambda i, be: (i, 0)),
            pl.BlockSpec((1, F, H), lambda i, be: (be[0, i], 0, 0)),
            pl.BlockSpec((1, F, H), lambda i, be: (be[0, i], 0, 0)),
            pl.BlockSpec((1, H, F), lambda i, be: (be[0, i], 0, 0)),
            pl.BlockSpec((BM, 128), lambda i, be: (i, 0)),
        ],
        out_specs=pl.BlockSpec((BM, H), lambda i, be: (i, 0)),
    )
    return pl.pallas_call(
        _experts_body,
        grid_spec=grid_spec,
        out_shape=jax.ShapeDtypeStruct((M, H), jnp.float32),
        compiler_params=pltpu.CompilerParams(
            dimension_semantics=("arbitrary",),
        ),
    )(block_expert, xs2, gate, up, down, ws)


def _combine(ys, pos):
    mesh = plsc.VectorSubcoreMesh(core_axis_name="c", subcore_axis_name="s")
    CH = 16
    NC = TPW // CH

    def k(ys_hbm, pos_hbm, out_hbm, i0_v, i1_v, b0a_v, b1a_v, b0b_v, b1b_v,
          sem_a, sem_b, sem_o):
        wid = lax.axis_index("s") * 2 + lax.axis_index("c")
        prow = wid // 2
        pcol = (wid % 2) * TPW
        pltpu.sync_copy(pos_hbm.at[prow, pl.ds(pcol, TPW)], i0_v)
        pltpu.sync_copy(pos_hbm.at[16 + prow, pl.ds(pcol, TPW)], i1_v)
        bufs = ((b0a_v, b1a_v, sem_a), (b0b_v, b1b_v, sem_b))

        def start(c, b0, b1, sem):
            pltpu.async_copy(ys_hbm.at[i0_v.at[pl.ds(c * CH, CH)]], b0, sem)
            pltpu.async_copy(ys_hbm.at[i1_v.at[pl.ds(c * CH, CH)]], b1, sem)

        start(0, *bufs[0])
        for c in range(NC):
            b0, b1, sem = bufs[c % 2]
            pltpu.make_async_copy(ys_hbm.at[pl.ds(0, CH)], b0, sem).wait()
            pltpu.make_async_copy(ys_hbm.at[pl.ds(0, CH)], b1, sem).wait()
            if c > 0:
                pltpu.make_async_copy(
                    bufs[(c - 1) % 2][0],
                    out_hbm.at[pl.ds(wid * TPW + (c - 1) * CH, CH)],
                    sem_o).wait()
            if c + 1 < NC:
                start(c + 1, *bufs[(c + 1) % 2])

            @pl.loop(0, CH)
            def _(r):
                for j in range(0, H, 16):
                    b0[r, pl.ds(j, 16)] = (
                        b0[r, pl.ds(j, 16)] + b1[r, pl.ds(j, 16)])

            pltpu.async_copy(
                b0, out_hbm.at[pl.ds(wid * TPW + c * CH, CH)], sem_o)
        pltpu.make_async_copy(
            bufs[(NC - 1) % 2][0],
            out_hbm.at[pl.ds(wid * TPW + (NC - 1) * CH, CH)], sem_o).wait()

    kk = pl.kernel(
        k,
        out_type=jax.ShapeDtypeStruct((N, H), jnp.float32),
        mesh=mesh,
        scratch_types=[
            pltpu.VMEM((TPW,), jnp.int32),
            pltpu.VMEM((TPW,), jnp.int32),
            pltpu.VMEM((CH, H), jnp.float32),
            pltpu.VMEM((CH, H), jnp.float32),
            pltpu.VMEM((CH, H), jnp.float32),
            pltpu.VMEM((CH, H), jnp.float32),
            pltpu.SemaphoreType.DMA,
            pltpu.SemaphoreType.DMA,
            pltpu.SemaphoreType.DMA,
        ],
    )
    return kk(ys, pos)


def kernel(hidden_states, router_weight, gate_proj, up_proj, down_proj):
    B, T, Hc = hidden_states.shape
    x = hidden_states.reshape(-1, Hc)
    pos, wr, be = _router(x, router_weight)
    xs, ws = _dispatch(x, wr, pos)
    ys = _experts(be, xs, gate_proj, up_proj, down_proj, ws)
    out = _combine(ys, pos)
    return out.reshape(B, T, Hc)
